# baseline (speedup 1.0000x reference)
import jax
import jax.numpy as jnp
from jax.experimental import pallas as pl
from jax.experimental.pallas import tpu as pltpu


def _fused_kernel(x_ref, w_ref, b_ref, o_ref):
    x = jnp.maximum(x_ref[...], 0.0)
    acc = jnp.dot(x, w_ref[...], preferred_element_type=jnp.float32)
    o_ref[...] = (acc + b_ref[...]).astype(o_ref.dtype)


@jax.jit
def kernel(feat, w, b):
    B, D = feat.shape
    _, N = w.shape

    tb = min(4096, max(8, (B + 7) // 8 * 8))
    b_pad = (B + tb - 1) // tb * tb
    feat_p = jnp.pad(feat, ((0, b_pad - B), (0, 512 - D)))
    w = jnp.pad(w, ((0, 512 - D), (0, 0)))
    D = 512

    out = pl.pallas_call(
        _fused_kernel,
        out_shape=jax.ShapeDtypeStruct((b_pad, N), feat.dtype),
        grid=(b_pad // tb,),
        in_specs=[
            pl.BlockSpec((tb, D), lambda i: (i, 0)),
            pl.BlockSpec((D, N), lambda i: (0, 0)),
            pl.BlockSpec((1, N), lambda i: (0, 0)),
        ],
        out_specs=pl.BlockSpec((tb, N), lambda i: (i, 0)),
        compiler_params=pltpu.CompilerParams(
            dimension_semantics=("parallel",),
        ),
    )(feat_p, w, b.reshape(1, N))

    return out[:B] if b_pad != B else out


# speedup vs baseline: 1.4587x; 1.4587x over previous
import jax
import jax.numpy as jnp
from jax.experimental import pallas as pl
from jax.experimental.pallas import tpu as pltpu

_N_PAD = 128


def _fused_kernel(x_ref, w_ref, b_ref, o_ref):
    x = jnp.maximum(x_ref[...], 0.0)
    acc = jnp.dot(x, w_ref[...], preferred_element_type=jnp.float32)
    o_ref[...] = (acc + b_ref[...]).astype(o_ref.dtype)


@jax.jit
def kernel(feat, w, b):
    B, D = feat.shape
    _, N = w.shape

    tb = min(4096, max(8, (B + 7) // 8 * 8))
    b_pad = (B + tb - 1) // tb * tb
    feat_p = jnp.pad(feat, ((0, b_pad - B), (0, 0))) if b_pad != B else feat
    w_p = jnp.pad(w, ((0, 0), (0, _N_PAD - N)))
    b_p = jnp.pad(b.reshape(1, N), ((0, 0), (0, _N_PAD - N)))

    out = pl.pallas_call(
        _fused_kernel,
        out_shape=jax.ShapeDtypeStruct((b_pad, _N_PAD), feat.dtype),
        grid=(b_pad // tb,),
        in_specs=[
            pl.BlockSpec((tb, D), lambda i: (i, 0)),
            pl.BlockSpec((D, _N_PAD), lambda i: (0, 0)),
            pl.BlockSpec((1, _N_PAD), lambda i: (0, 0)),
        ],
        out_specs=pl.BlockSpec((tb, _N_PAD), lambda i: (i, 0)),
        compiler_params=pltpu.CompilerParams(
            dimension_semantics=("parallel",),
        ),
    )(feat_p, w_p, b_p)

    return out[:B, :N]


# speedup vs baseline: 1.4603x; 1.0011x over previous
import functools

import jax
import jax.numpy as jnp
from jax import lax
from jax.experimental import pallas as pl
from jax.experimental.pallas import tpu as pltpu

_N_PAD = 128
_N_SLOTS = 6


def _ring_kernel(x_hbm, w_ref, b_ref, o_hbm, x_buf, o_buf, in_sem, out_sem,
                 *, block, n_steps):
    def dma_in(slot, step):
        pltpu.make_async_copy(
            x_hbm.at[pl.ds(step * block, block), :],
            x_buf.at[slot], in_sem.at[slot]).start()

    def wait_in(slot):
        pltpu.make_async_copy(
            x_hbm.at[pl.ds(0, block), :],
            x_buf.at[slot], in_sem.at[slot]).wait()

    def dma_out(slot, step):
        pltpu.make_async_copy(
            o_buf.at[slot],
            o_hbm.at[pl.ds(step * block, block), :], out_sem.at[slot]).start()

    def wait_out(slot):
        pltpu.make_async_copy(
            o_buf.at[0],
            o_hbm.at[pl.ds(0, block), :], out_sem.at[slot]).wait()

    for s in range(min(_N_SLOTS, n_steps)):
        dma_in(s, s)

    def body(step, _):
        slot = lax.rem(step, _N_SLOTS)
        wait_in(slot)
        oslot = lax.rem(step, 2)

        @pl.when(step >= 2)
        def _():
            wait_out(oslot)

        x = jnp.maximum(x_buf[slot], 0.0)
        acc = jnp.dot(x, w_ref[...], preferred_element_type=jnp.float32)
        o_buf[oslot] = acc + b_ref[...]
        dma_out(oslot, step)

        nxt = step + _N_SLOTS

        @pl.when(nxt < n_steps)
        def _():
            dma_in(slot, nxt)

        return ()

    lax.fori_loop(0, n_steps, body, ())

    @pl.when(n_steps >= 2)
    def _():
        wait_out(lax.rem(n_steps - 2, 2))

    wait_out(lax.rem(n_steps - 1, 2))


@jax.jit
def kernel(feat, w, b):
    B, D = feat.shape
    _, N = w.shape

    block = min(1024, max(8, (B + 7) // 8 * 8))
    b_pad = (B + block - 1) // block * block
    feat_p = jnp.pad(feat, ((0, b_pad - B), (0, 0))) if b_pad != B else feat
    w_p = jnp.pad(w, ((0, 0), (0, _N_PAD - N)))
    b_p = jnp.pad(b.reshape(1, N), ((0, 0), (0, _N_PAD - N)))
    n_steps = b_pad // block

    out = pl.pallas_call(
        functools.partial(_ring_kernel, block=block, n_steps=n_steps),
        out_shape=jax.ShapeDtypeStruct((b_pad, _N_PAD), feat.dtype),
        in_specs=[
            pl.BlockSpec(memory_space=pl.ANY),
            pl.BlockSpec(memory_space=pltpu.VMEM),
            pl.BlockSpec(memory_space=pltpu.VMEM),
        ],
        out_specs=pl.BlockSpec(memory_space=pl.ANY),
        scratch_shapes=[
            pltpu.VMEM((_N_SLOTS, block, D), jnp.float32),
            pltpu.VMEM((2, block, _N_PAD), jnp.float32),
            pltpu.SemaphoreType.DMA((_N_SLOTS,)),
            pltpu.SemaphoreType.DMA((2,)),
        ],
    )(feat_p, w_p, b_p)

    return out[:B, :N]
